# baseline (speedup 1.0000x reference)
import functools

import jax
import jax.numpy as jnp
from jax import lax
from jax.experimental import pallas as pl
from jax.experimental.pallas import tpu as pltpu
from jax.experimental.pallas import tpu_sc as plsc

N = 100000
E = 1600000
G = 1024
F_XS, F_XT, F_E, F_U, F_OUT = 10, 5, 10, 10, 5

PAD = 16
NC, NS = 2, 16
NW = NC * NS
EW = E // NW
CHUNK = 1000

BN = 1000
BE = 4000


def _project_nodes_body(xs_ref, xt_ref, w1_ref, ps_ref, pt_ref):
    w = w1_ref[...]
    ps = jnp.dot(xs_ref[...], w[0:F_XS], preferred_element_type=jnp.float32)
    pt = jnp.dot(xt_ref[...], w[F_XS:F_XS + F_XT],
                 preferred_element_type=jnp.float32)
    z = jnp.zeros((ps.shape[0], PAD - F_OUT), jnp.float32)
    ps_ref[...] = jnp.concatenate([ps, z], axis=1)
    pt_ref[...] = jnp.concatenate([pt, z], axis=1)


def _project_nodes(x_s, x_t, w1):
    grid = N // BN
    return pl.pallas_call(
        _project_nodes_body,
        grid=(grid,),
        in_specs=[
            pl.BlockSpec((BN, F_XS), lambda i: (i, 0)),
            pl.BlockSpec((BN, F_XT), lambda i: (i, 0)),
            pl.BlockSpec(w1.shape, lambda i: (0, 0)),
        ],
        out_specs=[
            pl.BlockSpec((BN, PAD), lambda i: (i, 0)),
            pl.BlockSpec((BN, PAD), lambda i: (i, 0)),
        ],
        out_shape=[
            jax.ShapeDtypeStruct((N, PAD), jnp.float32),
            jax.ShapeDtypeStruct((N, PAD), jnp.float32),
        ],
    )(x_s, x_t, w1)


def _project_globals_body(u_ref, w1_ref, b1_ref, pu_ref):
    w = w1_ref[...]
    pu = jnp.dot(u_ref[...], w[F_XS + F_XT + F_E:],
                 preferred_element_type=jnp.float32) + b1_ref[...]
    z = jnp.zeros((G, PAD - F_OUT), jnp.float32)
    pu_ref[...] = jnp.concatenate([pu, z], axis=1)


def _project_globals(u, w1, b1):
    return pl.pallas_call(
        _project_globals_body,
        out_shape=jax.ShapeDtypeStruct((G, PAD), jnp.float32),
    )(u, w1, b1.reshape(1, F_OUT))


def _sc_gather_body(ps_hbm, pt_hbm, pu_hbm, src_hbm, tgt_hbm, be_hbm, s_hbm,
                    src_v, tgt_v, be_v, rs_v, rt_v, ru_v, sem_s, sem_t, sem_u):
    wid = lax.axis_index("s") * NC + lax.axis_index("c")

    def chunk_body(ci, carry):
        base = wid * EW + ci * CHUNK
        pltpu.sync_copy(src_hbm.at[pl.ds(base, CHUNK)], src_v)
        pltpu.sync_copy(tgt_hbm.at[pl.ds(base, CHUNK)], tgt_v)
        pltpu.sync_copy(be_hbm.at[pl.ds(base, CHUNK)], be_v)
        cp_s = pltpu.async_copy(ps_hbm.at[src_v], rs_v, sem_s)
        cp_t = pltpu.async_copy(pt_hbm.at[tgt_v], rt_v, sem_t)
        cp_u = pltpu.async_copy(pu_hbm.at[be_v], ru_v, sem_u)
        cp_s.wait()
        cp_t.wait()
        cp_u.wait()

        def add_body(e, c2):
            rs_v[e, :] = rs_v[e, :] + rt_v[e, :] + ru_v[e, :]
            return c2

        lax.fori_loop(0, CHUNK, add_body, 0, unroll=8)
        pltpu.sync_copy(rs_v.at[:, pl.ds(0, F_OUT)],
                        s_hbm.at[pl.ds(base, CHUNK)])
        return carry

    lax.fori_loop(0, EW // CHUNK, chunk_body, 0)


def _sc_gather(ps, pt, pu, src, tgt, be):
    kern = functools.partial(
        pl.kernel,
        out_type=jax.ShapeDtypeStruct((E, F_OUT), jnp.float32),
        mesh=plsc.VectorSubcoreMesh(core_axis_name="c", subcore_axis_name="s"),
        compiler_params=pltpu.CompilerParams(use_tc_tiling_on_sc=False),
        scratch_types=[
            pltpu.VMEM((CHUNK,), jnp.int32),
            pltpu.VMEM((CHUNK,), jnp.int32),
            pltpu.VMEM((CHUNK,), jnp.int32),
            pltpu.VMEM((CHUNK, PAD), jnp.float32),
            pltpu.VMEM((CHUNK, PAD), jnp.float32),
            pltpu.VMEM((CHUNK, PAD), jnp.float32),
            pltpu.SemaphoreType.DMA,
            pltpu.SemaphoreType.DMA,
            pltpu.SemaphoreType.DMA,
        ],
    )(_sc_gather_body)
    return kern(ps, pt, pu, src, tgt, be)


R = 64
ROWS = E // R
BR = 200


def _epilogue_body(s_ref, ea_ref, bd1_ref, bd2_ref, b2_ref, o_ref):
    d = jnp.dot(ea_ref[...], bd1_ref[...], preferred_element_type=jnp.float32)
    h = s_ref[...] + d
    h = jnp.where(h > 0, h, 0.1 * h)
    o_ref[...] = jnp.dot(h, bd2_ref[...],
                         preferred_element_type=jnp.float32) + b2_ref[...]


def _epilogue(s, edge_attr, w1e, w2, b2):
    s_p = s.reshape(ROWS, R * F_OUT)
    ea_p = edge_attr.reshape(ROWS, R * F_E)
    eye = jnp.eye(R, dtype=jnp.float32)
    bd1 = jnp.kron(eye, w1e)
    bd2 = jnp.kron(eye, w2)
    b2_t = jnp.tile(b2, R).reshape(1, R * F_OUT)
    out_p = pl.pallas_call(
        _epilogue_body,
        grid=(ROWS // BR,),
        in_specs=[
            pl.BlockSpec((BR, R * F_OUT), lambda i: (i, 0)),
            pl.BlockSpec((BR, R * F_E), lambda i: (i, 0)),
            pl.BlockSpec((R * F_E, R * F_OUT), lambda i: (0, 0)),
            pl.BlockSpec((R * F_OUT, R * F_OUT), lambda i: (0, 0)),
            pl.BlockSpec((1, R * F_OUT), lambda i: (0, 0)),
        ],
        out_specs=pl.BlockSpec((BR, R * F_OUT), lambda i: (i, 0)),
        out_shape=jax.ShapeDtypeStruct((ROWS, R * F_OUT), jnp.float32),
    )(s_p, ea_p, bd1, bd2, b2_t)
    return out_p.reshape(E, F_OUT)


def kernel(x_s, x_t, edge_index, edge_attr, u, batch_e, W1, b1, W2, b2):
    src = edge_index[0]
    tgt = edge_index[1]
    w1e = W1[F_XS + F_XT:F_XS + F_XT + F_E]
    ps, pt = _project_nodes(x_s, x_t, W1)
    pu = _project_globals(u, W1, b1)
    s = _sc_gather(ps, pt, pu, src, tgt, batch_e)
    return _epilogue(s, edge_attr, w1e, W2, b2)


# speedup vs baseline: 2.3361x; 2.3361x over previous
import functools

import jax
import jax.numpy as jnp
from jax import lax
from jax.experimental import pallas as pl
from jax.experimental.pallas import tpu as pltpu
from jax.experimental.pallas import tpu_sc as plsc

N = 100000
E = 1600000
G = 1024
F_XS, F_XT, F_E, F_U, F_OUT = 10, 5, 10, 10, 5

PAD = 8
NC, NS = 2, 16
NW = NC * NS
EW = E // NW
CHUNK = 1000

BN = 1000


def _project_nodes_body(xs_ref, xt_ref, w1_ref, ps_ref, pt_ref):
    w = w1_ref[...]
    ps = jnp.dot(xs_ref[...], w[0:F_XS], preferred_element_type=jnp.float32)
    pt = jnp.dot(xt_ref[...], w[F_XS:F_XS + F_XT],
                 preferred_element_type=jnp.float32)
    z = jnp.zeros((ps.shape[0], PAD - F_OUT), jnp.float32)
    ps_ref[...] = jnp.concatenate([ps, z], axis=1)
    pt_ref[...] = jnp.concatenate([pt, z], axis=1)


def _project_nodes(x_s, x_t, w1):
    grid = N // BN
    return pl.pallas_call(
        _project_nodes_body,
        grid=(grid,),
        in_specs=[
            pl.BlockSpec((BN, F_XS), lambda i: (i, 0)),
            pl.BlockSpec((BN, F_XT), lambda i: (i, 0)),
            pl.BlockSpec(w1.shape, lambda i: (0, 0)),
        ],
        out_specs=[
            pl.BlockSpec((BN, PAD), lambda i: (i, 0)),
            pl.BlockSpec((BN, PAD), lambda i: (i, 0)),
        ],
        out_shape=[
            jax.ShapeDtypeStruct((N, PAD), jnp.float32),
            jax.ShapeDtypeStruct((N, PAD), jnp.float32),
        ],
    )(x_s, x_t, w1)


def _project_globals_body(u_ref, w1_ref, b1_ref, pu_ref):
    w = w1_ref[...]
    pu = jnp.dot(u_ref[...], w[F_XS + F_XT + F_E:],
                 preferred_element_type=jnp.float32) + b1_ref[...]
    z = jnp.zeros((G, PAD - F_OUT), jnp.float32)
    pu_ref[...] = jnp.concatenate([pu, z], axis=1)


def _project_globals(u, w1, b1):
    return pl.pallas_call(
        _project_globals_body,
        out_shape=jax.ShapeDtypeStruct((G, PAD), jnp.float32),
    )(u, w1, b1.reshape(1, F_OUT))


def _sc_gather_body(ps_hbm, pt_hbm, pu_hbm, src_hbm, tgt_hbm, be_hbm,
                    gs_hbm, gt_hbm, gu_hbm,
                    src_v, tgt_v, be_v, rs_v, rt_v, ru_v,
                    sem_s, sem_t, sem_u):
    wid = lax.axis_index("s") * NC + lax.axis_index("c")

    def chunk_body(ci, carry):
        base = wid * EW + ci * CHUNK
        pltpu.sync_copy(src_hbm.at[pl.ds(base, CHUNK)], src_v)
        pltpu.sync_copy(tgt_hbm.at[pl.ds(base, CHUNK)], tgt_v)
        pltpu.sync_copy(be_hbm.at[pl.ds(base, CHUNK)], be_v)
        cp_s = pltpu.async_copy(ps_hbm.at[src_v], rs_v, sem_s)
        cp_t = pltpu.async_copy(pt_hbm.at[tgt_v], rt_v, sem_t)
        cp_u = pltpu.async_copy(pu_hbm.at[be_v], ru_v, sem_u)
        cp_s.wait()
        cp_t.wait()
        cp_u.wait()
        pltpu.sync_copy(rs_v, gs_hbm.at[pl.ds(base, CHUNK)])
        pltpu.sync_copy(rt_v, gt_hbm.at[pl.ds(base, CHUNK)])
        pltpu.sync_copy(ru_v, gu_hbm.at[pl.ds(base, CHUNK)])
        return carry

    lax.fori_loop(0, EW // CHUNK, chunk_body, 0)


def _sc_gather(ps, pt, pu, src, tgt, be):
    kern = functools.partial(
        pl.kernel,
        out_type=(
            jax.ShapeDtypeStruct((E, PAD), jnp.float32),
            jax.ShapeDtypeStruct((E, PAD), jnp.float32),
            jax.ShapeDtypeStruct((E, PAD), jnp.float32),
        ),
        mesh=plsc.VectorSubcoreMesh(core_axis_name="c", subcore_axis_name="s"),
        compiler_params=pltpu.CompilerParams(use_tc_tiling_on_sc=False),
        scratch_types=[
            pltpu.VMEM((CHUNK,), jnp.int32),
            pltpu.VMEM((CHUNK,), jnp.int32),
            pltpu.VMEM((CHUNK,), jnp.int32),
            pltpu.VMEM((CHUNK, PAD), jnp.float32),
            pltpu.VMEM((CHUNK, PAD), jnp.float32),
            pltpu.VMEM((CHUNK, PAD), jnp.float32),
            pltpu.SemaphoreType.DMA,
            pltpu.SemaphoreType.DMA,
            pltpu.SemaphoreType.DMA,
        ],
    )(_sc_gather_body)
    return kern(ps, pt, pu, src, tgt, be)


R = 64
ROWS = E // R
BR = 200


def _epilogue_body(gs_ref, gt_ref, gu_ref, ea_ref, sel_ref, bd1_ref,
                   bd2_ref, b2_ref, o_ref):
    g = gs_ref[...] + gt_ref[...] + gu_ref[...]
    hs = jnp.dot(g, sel_ref[...], preferred_element_type=jnp.float32)
    d = jnp.dot(ea_ref[...], bd1_ref[...], preferred_element_type=jnp.float32)
    h = hs + d
    h = jnp.where(h > 0, h, 0.1 * h)
    o_ref[...] = jnp.dot(h, bd2_ref[...],
                         preferred_element_type=jnp.float32) + b2_ref[...]


def _epilogue(gs, gt, gu, edge_attr, w1e, w2, b2):
    gs_p = gs.reshape(ROWS, R * PAD)
    gt_p = gt.reshape(ROWS, R * PAD)
    gu_p = gu.reshape(ROWS, R * PAD)
    ea_p = edge_attr.reshape(ROWS, R * F_E)
    eye = jnp.eye(R, dtype=jnp.float32)
    p85 = jnp.zeros((PAD, F_OUT), jnp.float32).at[:F_OUT, :].set(
        jnp.eye(F_OUT, dtype=jnp.float32))
    sel = jnp.kron(eye, p85)
    bd1 = jnp.kron(eye, w1e)
    bd2 = jnp.kron(eye, w2)
    b2_t = jnp.tile(b2, R).reshape(1, R * F_OUT)
    out_p = pl.pallas_call(
        _epilogue_body,
        grid=(ROWS // BR,),
        in_specs=[
            pl.BlockSpec((BR, R * PAD), lambda i: (i, 0)),
            pl.BlockSpec((BR, R * PAD), lambda i: (i, 0)),
            pl.BlockSpec((BR, R * PAD), lambda i: (i, 0)),
            pl.BlockSpec((BR, R * F_E), lambda i: (i, 0)),
            pl.BlockSpec((R * PAD, R * F_OUT), lambda i: (0, 0)),
            pl.BlockSpec((R * F_E, R * F_OUT), lambda i: (0, 0)),
            pl.BlockSpec((R * F_OUT, R * F_OUT), lambda i: (0, 0)),
            pl.BlockSpec((1, R * F_OUT), lambda i: (0, 0)),
        ],
        out_specs=pl.BlockSpec((BR, R * F_OUT), lambda i: (i, 0)),
        out_shape=jax.ShapeDtypeStruct((ROWS, R * F_OUT), jnp.float32),
    )(gs_p, gt_p, gu_p, ea_p, sel, bd1, bd2, b2_t)
    return out_p.reshape(E, F_OUT)


def kernel(x_s, x_t, edge_index, edge_attr, u, batch_e, W1, b1, W2, b2):
    src = edge_index[0]
    tgt = edge_index[1]
    w1e = W1[F_XS + F_XT:F_XS + F_XT + F_E]
    ps, pt = _project_nodes(x_s, x_t, W1)
    pu = _project_globals(u, W1, b1)
    gs, gt, gu = _sc_gather(ps, pt, pu, src, tgt, batch_e)
    return _epilogue(gs, gt, gu, edge_attr, w1e, W2, b2)


# speedup vs baseline: 2.4616x; 1.0537x over previous
import functools

import jax
import jax.numpy as jnp
from jax import lax
from jax.experimental import pallas as pl
from jax.experimental.pallas import tpu as pltpu
from jax.experimental.pallas import tpu_sc as plsc

N = 100000
E = 1600000
G = 1024
F_XS, F_XT, F_E, F_U, F_OUT = 10, 5, 10, 10, 5

PAD = 8
NC, NS = 2, 16
NW = NC * NS
EW = E // NW
CHUNK = 1000

BN = 2048
NP = 102400


def _project_nodes_body(xst_ref, xtt_ref, w1_ref, ps_ref, pt_ref):
    w = w1_ref[...]
    dn = (((0,), (0,)), ((), ()))
    ps = lax.dot_general(xst_ref[...], w[0:F_XS], dn,
                         preferred_element_type=jnp.float32)
    pt = lax.dot_general(xtt_ref[...], w[F_XS:F_XS + F_XT], dn,
                         preferred_element_type=jnp.float32)
    z = jnp.zeros((BN, PAD - F_OUT), jnp.float32)
    ps_ref[...] = jnp.concatenate([ps, z], axis=1)
    pt_ref[...] = jnp.concatenate([pt, z], axis=1)


def _project_nodes(x_st, x_tt, w1):
    x_st = jnp.pad(x_st, ((0, 0), (0, NP - N)))
    x_tt = jnp.pad(x_tt, ((0, 0), (0, NP - N)))
    return pl.pallas_call(
        _project_nodes_body,
        grid=(NP // BN,),
        in_specs=[
            pl.BlockSpec((F_XS, BN), lambda i: (0, i)),
            pl.BlockSpec((F_XT, BN), lambda i: (0, i)),
            pl.BlockSpec(w1.shape, lambda i: (0, 0)),
        ],
        out_specs=[
            pl.BlockSpec((BN, PAD), lambda i: (i, 0)),
            pl.BlockSpec((BN, PAD), lambda i: (i, 0)),
        ],
        out_shape=[
            jax.ShapeDtypeStruct((NP, PAD), jnp.float32),
            jax.ShapeDtypeStruct((NP, PAD), jnp.float32),
        ],
    )(x_st, x_tt, w1)


def _project_globals_body(ut_ref, w1_ref, b1_ref, pu_ref):
    w = w1_ref[...]
    dn = (((0,), (0,)), ((), ()))
    pu = lax.dot_general(ut_ref[...], w[F_XS + F_XT + F_E:], dn,
                         preferred_element_type=jnp.float32) + b1_ref[...]
    z = jnp.zeros((G, PAD - F_OUT), jnp.float32)
    pu_ref[...] = jnp.concatenate([pu, z], axis=1)


def _project_globals(ut, w1, b1):
    return pl.pallas_call(
        _project_globals_body,
        out_shape=jax.ShapeDtypeStruct((G, PAD), jnp.float32),
    )(ut, w1, b1.reshape(1, F_OUT))


def _sc_gather_body(ps_hbm, pt_hbm, pu_hbm, ei_hbm, be_hbm,
                    gs_hbm, gt_hbm, gu_hbm,
                    src_v, tgt_v, be_v, rs_v, rt_v, ru_v,
                    sem_s, sem_t, sem_u):
    wid = lax.axis_index("s") * NC + lax.axis_index("c")

    def chunk_body(ci, carry):
        base = wid * EW + ci * CHUNK
        pltpu.sync_copy(ei_hbm.at[0, pl.ds(base, CHUNK)], src_v)
        pltpu.sync_copy(ei_hbm.at[1, pl.ds(base, CHUNK)], tgt_v)
        pltpu.sync_copy(be_hbm.at[pl.ds(base, CHUNK)], be_v)
        cp_s = pltpu.async_copy(ps_hbm.at[src_v], rs_v, sem_s)
        cp_t = pltpu.async_copy(pt_hbm.at[tgt_v], rt_v, sem_t)
        cp_u = pltpu.async_copy(pu_hbm.at[be_v], ru_v, sem_u)
        cp_s.wait()
        cp_t.wait()
        cp_u.wait()
        pltpu.sync_copy(rs_v, gs_hbm.at[pl.ds(base, CHUNK)])
        pltpu.sync_copy(rt_v, gt_hbm.at[pl.ds(base, CHUNK)])
        pltpu.sync_copy(ru_v, gu_hbm.at[pl.ds(base, CHUNK)])
        return carry

    lax.fori_loop(0, EW // CHUNK, chunk_body, 0)


def _sc_gather(ps, pt, pu, edge_index, be):
    kern = functools.partial(
        pl.kernel,
        out_type=(
            jax.ShapeDtypeStruct((E, PAD), jnp.float32),
            jax.ShapeDtypeStruct((E, PAD), jnp.float32),
            jax.ShapeDtypeStruct((E, PAD), jnp.float32),
        ),
        mesh=plsc.VectorSubcoreMesh(core_axis_name="c", subcore_axis_name="s"),
        compiler_params=pltpu.CompilerParams(use_tc_tiling_on_sc=False),
        scratch_types=[
            pltpu.VMEM((CHUNK,), jnp.int32),
            pltpu.VMEM((CHUNK,), jnp.int32),
            pltpu.VMEM((CHUNK,), jnp.int32),
            pltpu.VMEM((CHUNK, PAD), jnp.float32),
            pltpu.VMEM((CHUNK, PAD), jnp.float32),
            pltpu.VMEM((CHUNK, PAD), jnp.float32),
            pltpu.SemaphoreType.DMA,
            pltpu.SemaphoreType.DMA,
            pltpu.SemaphoreType.DMA,
        ],
    )(_sc_gather_body)
    return kern(ps, pt, pu, edge_index, be)


R = 64
ROWS = E // R
BR = 200


def _epilogue_body(gs_ref, gt_ref, gu_ref, ea_ref, sel_ref, bd1_ref,
                   bd2_ref, b2_ref, o_ref):
    g = gs_ref[...] + gt_ref[...] + gu_ref[...]
    hs = jnp.dot(g, sel_ref[...], preferred_element_type=jnp.float32)
    d = jnp.dot(ea_ref[...], bd1_ref[...], preferred_element_type=jnp.float32)
    h = hs + d
    h = jnp.where(h > 0, h, 0.1 * h)
    o_ref[...] = jnp.dot(h, bd2_ref[...],
                         preferred_element_type=jnp.float32) + b2_ref[...]


def _epilogue(gs, gt, gu, edge_attr, w1e, w2, b2):
    gs_p = gs.reshape(ROWS, R * PAD)
    gt_p = gt.reshape(ROWS, R * PAD)
    gu_p = gu.reshape(ROWS, R * PAD)
    ea_p = edge_attr.reshape(ROWS, R * F_E)
    eye = jnp.eye(R, dtype=jnp.float32)
    p85 = jnp.zeros((PAD, F_OUT), jnp.float32).at[:F_OUT, :].set(
        jnp.eye(F_OUT, dtype=jnp.float32))
    sel = jnp.kron(eye, p85)
    bd1 = jnp.kron(eye, w1e)
    bd2 = jnp.kron(eye, w2)
    b2_t = jnp.tile(b2, R).reshape(1, R * F_OUT)
    out_p = pl.pallas_call(
        _epilogue_body,
        grid=(ROWS // BR,),
        in_specs=[
            pl.BlockSpec((BR, R * PAD), lambda i: (i, 0)),
            pl.BlockSpec((BR, R * PAD), lambda i: (i, 0)),
            pl.BlockSpec((BR, R * PAD), lambda i: (i, 0)),
            pl.BlockSpec((BR, R * F_E), lambda i: (i, 0)),
            pl.BlockSpec((R * PAD, R * F_OUT), lambda i: (0, 0)),
            pl.BlockSpec((R * F_E, R * F_OUT), lambda i: (0, 0)),
            pl.BlockSpec((R * F_OUT, R * F_OUT), lambda i: (0, 0)),
            pl.BlockSpec((1, R * F_OUT), lambda i: (0, 0)),
        ],
        out_specs=pl.BlockSpec((BR, R * F_OUT), lambda i: (i, 0)),
        out_shape=jax.ShapeDtypeStruct((ROWS, R * F_OUT), jnp.float32),
    )(gs_p, gt_p, gu_p, ea_p, sel, bd1, bd2, b2_t)
    return out_p.reshape(E, F_OUT)


def kernel(x_s, x_t, edge_index, edge_attr, u, batch_e, W1, b1, W2, b2):
    w1e = W1[F_XS + F_XT:F_XS + F_XT + F_E]
    ps, pt = _project_nodes(x_s.T, x_t.T, W1)
    pu = _project_globals(u.T, W1, b1)
    gs, gt, gu = _sc_gather(ps, pt, pu, edge_index, batch_e)
    return _epilogue(gs, gt, gu, edge_attr, w1e, W2, b2)
